# baseline (speedup 1.0000x reference)
import jax
import jax.numpy as jnp
from jax import lax
from jax.experimental import pallas as pl
from jax.experimental.pallas import tpu as pltpu

N = 50000
B = 128
D = 256
BLK = 10000
PAD127 = 48
ROWS = (N + PAD127) // 128 + 1


def _fused_kernel(x_ref, bfull_ref, out_ref, se_ref, inv_ref):
    @pl.when(pl.program_id(0) == 0)
    def _compute_bounds():
        A = bfull_ref[...]
        lane = lax.broadcasted_iota(jnp.int32, (1, B), 1)
        rowmax = jnp.max(A, axis=1, keepdims=True)
        full = jnp.sum((rowmax <= lane).astype(jnp.float32), axis=0,
                       keepdims=True)
        eye = (
            lax.broadcasted_iota(jnp.int32, (B, B), 0)
            == lax.broadcasted_iota(jnp.int32, (B, B), 1)
        ).astype(jnp.float32)
        full_t = lax.dot_general(
            eye, full, (((1,), (1,)), ((), ())),
            preferred_element_type=jnp.float32,
            precision=lax.Precision.HIGHEST,
        )
        rowid = lax.broadcasted_iota(jnp.int32, (B, ROWS), 1).astype(
            jnp.float32)
        ohsel = (rowid == full_t).astype(jnp.float32)
        selrow = lax.dot_general(
            ohsel, A.astype(jnp.float32), (((1,), (0,)), ((), ())),
            preferred_element_type=jnp.float32,
            precision=lax.Precision.HIGHEST,
        )
        bsub = lax.broadcasted_iota(jnp.int32, (B, 1), 0).astype(
            jnp.float32)
        part = jnp.sum((selrow <= bsub).astype(jnp.float32), axis=1,
                       keepdims=True)
        cnt_sub = full_t * 128.0 + part
        ends_f = lax.dot_general(
            cnt_sub, eye, (((0,), (0,)), ((), ())),
            preferred_element_type=jnp.float32,
            precision=lax.Precision.HIGHEST,
        )
        shift = (
            lax.broadcasted_iota(jnp.int32, (B, B), 0)
            == lax.broadcasted_iota(jnp.int32, (B, B), 1) - 1
        ).astype(jnp.float32)
        starts_f = lax.dot_general(
            cnt_sub, shift, (((0,), (0,)), ((), ())),
            preferred_element_type=jnp.float32,
            precision=lax.Precision.HIGHEST,
        )
        ends = ends_f.astype(jnp.int32)
        ends = jnp.where(lane == B - 1, N, ends)
        starts = starts_f.astype(jnp.int32)
        deg_f = (ends - starts).astype(jnp.float32)
        se_ref[0:1, :] = starts
        se_ref[1:2, :] = ends
        inv_ref[...] = lax.rsqrt(jnp.maximum(deg_f, 1.0))

    i = pl.program_id(0)
    rows = lax.broadcasted_iota(jnp.int32, (BLK, B), 0) + i * BLK
    oh = (
        (rows >= se_ref[0:1, :]) & (rows < se_ref[1:2, :])
    ).astype(jnp.float32)
    scale = lax.dot_general(
        oh, inv_ref[...], (((1,), (1,)), ((), ())),
        preferred_element_type=jnp.float32,
        precision=lax.Precision.HIGHEST,
    )
    out_ref[...] = x_ref[...] * scale


def kernel(x, batch):
    b32 = batch.astype(jnp.int32)
    bfull = jnp.concatenate([
        b32,
        jnp.full((PAD127,), B - 1, jnp.int32),
        jnp.full((B,), B, jnp.int32),
    ]).reshape(ROWS, 128)

    out = pl.pallas_call(
        _fused_kernel,
        grid=(N // BLK,),
        in_specs=[
            pl.BlockSpec((BLK, D), lambda i: (i, 0)),
            pl.BlockSpec((ROWS, 128), lambda i: (0, 0)),
        ],
        out_specs=pl.BlockSpec((BLK, D), lambda i: (i, 0)),
        out_shape=jax.ShapeDtypeStruct(x.shape, x.dtype),
        scratch_shapes=[
            pltpu.VMEM((2, B), jnp.int32),
            pltpu.VMEM((1, B), jnp.float32),
        ],
    )(x, bfull)
    return out
